# baseline (speedup 1.0000x reference)
import dataclasses
import functools

import jax
import jax.numpy as jnp
from jax import lax
from jax.experimental import pallas as pl
from jax.experimental.pallas import tpu as pltpu
from jax.experimental.pallas import tpu_sc as plsc

_GROUPS = (
    (0, 1, 2, 8, 25),
    (3, 24),
    (6, 7),
    (4, 9),
    (12, 13, 14, 15),
    (16, 17, 18, 19, 20, 21, 22, 23),
)

_N_COLS = 26
_N_OUT = 6
_LANES = 16
_ROWS_PER_BLOCK = 1600


def _block_body(in_vmem, out_vmem):
    lanes = lax.iota(jnp.int32, _LANES)
    zeros = lanes * 0

    @pl.loop(0, _ROWS_PER_BLOCK // _LANES)
    def _(i):
        rows = lanes + i * _LANES
        cache = {}

        def col(c):
            if c not in cache:
                cache[c] = plsc.load_gather(in_vmem, [rows, zeros + c])
            return cache[c]

        for g, idx in enumerate(_GROUPS):
            m = col(idx[0])
            for c in idx[1:]:
                m = jnp.maximum(m, col(c))
            plsc.store_scatter(out_vmem, [rows, zeros + g], m)


def kernel(activation):
    n_rows = activation.shape[0]
    n_blocks = n_rows // _ROWS_PER_BLOCK
    mesh = plsc.VectorSubcoreMesh(core_axis_name="c", subcore_axis_name="s")
    cp = pltpu.CompilerParams()
    if "needs_layout_passes" in pltpu.CompilerParams.__dataclass_fields__:
        cp = dataclasses.replace(cp, needs_layout_passes=False)
    if "use_tc_tiling_on_sc" in pltpu.CompilerParams.__dataclass_fields__:
        cp = dataclasses.replace(cp, use_tc_tiling_on_sc=False)

    @functools.partial(
        pl.kernel,
        out_type=jax.ShapeDtypeStruct((n_rows, _N_OUT), jnp.float32),
        mesh=mesh,
        compiler_params=cp,
    )
    def run(in_hbm, out_hbm):
        pltpu.emit_pipeline(
            _block_body,
            grid=(n_blocks,),
            in_specs=[
                pl.BlockSpec((_ROWS_PER_BLOCK, _N_COLS), lambda i: (i, 0))
            ],
            out_specs=[
                pl.BlockSpec((_ROWS_PER_BLOCK, _N_OUT), lambda i: (i, 0))
            ],
            core_axis_name=("c", "s"),
            dimension_semantics=(pltpu.PARALLEL,),
        )(in_hbm, out_hbm)

    return run(activation)


# speedup vs baseline: 1.6600x; 1.6600x over previous
import dataclasses
import functools

import jax
import jax.numpy as jnp
from jax import lax
from jax.experimental import pallas as pl
from jax.experimental.pallas import tpu as pltpu
from jax.experimental.pallas import tpu_sc as plsc

_GROUPS = (
    (0, 1, 2, 8, 25),
    (3, 24),
    (6, 7),
    (4, 9),
    (12, 13, 14, 15),
    (16, 17, 18, 19, 20, 21, 22, 23),
)

_N_COLS = 26
_N_OUT = 6
_LANES = 16
_ROWS_PER_BLOCK = 160


def _block_body(in_vmem, out_vmem):
    lanes = lax.iota(jnp.int32, _LANES)
    zeros = lanes * 0

    @pl.loop(0, _ROWS_PER_BLOCK // _LANES)
    def _(i):
        rows = lanes + i * _LANES
        cache = {}

        def col(c):
            if c not in cache:
                cache[c] = plsc.load_gather(in_vmem, [rows, zeros + c])
            return cache[c]

        for g, idx in enumerate(_GROUPS):
            m = col(idx[0])
            for c in idx[1:]:
                m = jnp.maximum(m, col(c))
            plsc.store_scatter(out_vmem, [rows, zeros + g], m)


def kernel(activation):
    n_rows = activation.shape[0]
    n_blocks = n_rows // _ROWS_PER_BLOCK
    mesh = plsc.VectorSubcoreMesh(core_axis_name="c", subcore_axis_name="s")
    cp = pltpu.CompilerParams()
    if "needs_layout_passes" in pltpu.CompilerParams.__dataclass_fields__:
        cp = dataclasses.replace(cp, needs_layout_passes=False)
    if "use_tc_tiling_on_sc" in pltpu.CompilerParams.__dataclass_fields__:
        cp = dataclasses.replace(cp, use_tc_tiling_on_sc=True)

    @functools.partial(
        pl.kernel,
        out_type=jax.ShapeDtypeStruct((n_rows, _N_OUT), jnp.float32),
        mesh=mesh,
        compiler_params=cp,
    )
    def run(in_hbm, out_hbm):
        pltpu.emit_pipeline(
            _block_body,
            grid=(n_blocks,),
            in_specs=[
                pl.BlockSpec((_ROWS_PER_BLOCK, _N_COLS), lambda i: (i, 0))
            ],
            out_specs=[
                pl.BlockSpec((_ROWS_PER_BLOCK, _N_OUT), lambda i: (i, 0))
            ],
            core_axis_name=("c", "s"),
            dimension_semantics=(pltpu.PARALLEL,),
        )(in_hbm, out_hbm)

    return run(activation)


# speedup vs baseline: 17.2306x; 10.3797x over previous
import dataclasses
import functools

import jax
import jax.numpy as jnp
from jax.experimental import pallas as pl
from jax.experimental.pallas import tpu as pltpu
from jax.experimental.pallas import tpu_sc as plsc

_GROUPS = (
    (0, 1, 2, 8, 25),
    (3, 24),
    (6, 7),
    (4, 9),
    (12, 13, 14, 15),
    (16, 17, 18, 19, 20, 21, 22, 23),
)

_N_COLS = 26
_N_OUT = 6
_LANES = 16
_BLOCK_W = 1024
_TILE = 128


def _block_body(in_vmem, out_vmem):
    @pl.loop(0, in_vmem.shape[1] // _LANES)
    def _(i):
        sl = pl.ds(i * _LANES, _LANES)
        for g, idx in enumerate(_GROUPS):
            m = in_vmem[idx[0], sl]
            for c in idx[1:]:
                m = jnp.maximum(m, in_vmem[c, sl])
            out_vmem[g, sl] = m


def kernel(activation):
    n_rows = activation.shape[0]
    n_tiles = -(-n_rows // _TILE)
    n_main = (n_tiles * _TILE) // _BLOCK_W
    tail_t0 = n_main * (_BLOCK_W // _TILE)
    n_tail = n_tiles - tail_t0
    act_t = activation.T
    mesh = plsc.VectorSubcoreMesh(core_axis_name="c", subcore_axis_name="s")
    cp = pltpu.CompilerParams()
    if "needs_layout_passes" in pltpu.CompilerParams.__dataclass_fields__:
        cp = dataclasses.replace(cp, needs_layout_passes=False)
    if "use_tc_tiling_on_sc" in pltpu.CompilerParams.__dataclass_fields__:
        cp = dataclasses.replace(cp, use_tc_tiling_on_sc=True)

    @functools.partial(
        pl.kernel,
        out_type=jax.ShapeDtypeStruct((_N_OUT, n_rows), jnp.float32),
        mesh=mesh,
        compiler_params=cp,
    )
    def run(in_hbm, out_hbm):
        pltpu.emit_pipeline(
            _block_body,
            grid=(n_main,),
            in_specs=[pl.BlockSpec((_N_COLS, _BLOCK_W), lambda i: (0, i))],
            out_specs=[pl.BlockSpec((_N_OUT, _BLOCK_W), lambda i: (0, i))],
            core_axis_name=("c", "s"),
            dimension_semantics=(pltpu.PARALLEL,),
        )(in_hbm, out_hbm)
        if n_tail:
            pltpu.emit_pipeline(
                _block_body,
                grid=(n_tail,),
                in_specs=[
                    pl.BlockSpec((_N_COLS, _TILE), lambda i: (0, i + tail_t0))
                ],
                out_specs=[
                    pl.BlockSpec((_N_OUT, _TILE), lambda i: (0, i + tail_t0))
                ],
                core_axis_name=("c", "s"),
                dimension_semantics=(pltpu.PARALLEL,),
            )(in_hbm, out_hbm)

    return run(act_t).T


# speedup vs baseline: 20.0492x; 1.1636x over previous
import dataclasses
import functools

import jax
import jax.numpy as jnp
from jax.experimental import pallas as pl
from jax.experimental.pallas import tpu as pltpu
from jax.experimental.pallas import tpu_sc as plsc

_GROUPS = (
    (0, 1, 2, 8, 25),
    (3, 24),
    (6, 7),
    (4, 9),
    (12, 13, 14, 15),
    (16, 17, 18, 19, 20, 21, 22, 23),
)

_N_COLS = 26
_N_OUT = 6
_LANES = 16
_BLOCK_W = 1024
_TILE = 128


def _block_body(in_vmem, out_vmem):
    @plsc.parallel_loop(0, in_vmem.shape[1] // _LANES, unroll=4)
    def _(i):
        sl = pl.ds(i * _LANES, _LANES)
        for g, idx in enumerate(_GROUPS):
            m = in_vmem[idx[0], sl]
            for c in idx[1:]:
                m = jnp.maximum(m, in_vmem[c, sl])
            out_vmem[g, sl] = m


def kernel(activation):
    n_rows = activation.shape[0]
    n_tiles = -(-n_rows // _TILE)
    n_main = (n_tiles * _TILE) // _BLOCK_W
    tail_t0 = n_main * (_BLOCK_W // _TILE)
    n_tail = n_tiles - tail_t0
    act_t = activation.T
    mesh = plsc.VectorSubcoreMesh(core_axis_name="c", subcore_axis_name="s")
    cp = pltpu.CompilerParams()
    if "needs_layout_passes" in pltpu.CompilerParams.__dataclass_fields__:
        cp = dataclasses.replace(cp, needs_layout_passes=False)
    if "use_tc_tiling_on_sc" in pltpu.CompilerParams.__dataclass_fields__:
        cp = dataclasses.replace(cp, use_tc_tiling_on_sc=True)

    @functools.partial(
        pl.kernel,
        out_type=jax.ShapeDtypeStruct((_N_OUT, n_rows), jnp.float32),
        mesh=mesh,
        compiler_params=cp,
    )
    def run(in_hbm, out_hbm):
        pltpu.emit_pipeline(
            _block_body,
            grid=(n_main,),
            in_specs=[pl.BlockSpec((_N_COLS, _BLOCK_W), lambda i: (0, i))],
            out_specs=[pl.BlockSpec((_N_OUT, _BLOCK_W), lambda i: (0, i))],
            core_axis_name=("c", "s"),
            dimension_semantics=(pltpu.PARALLEL,),
        )(in_hbm, out_hbm)
        if n_tail:
            pltpu.emit_pipeline(
                _block_body,
                grid=(n_tail,),
                in_specs=[
                    pl.BlockSpec((_N_COLS, _TILE), lambda i: (0, i + tail_t0))
                ],
                out_specs=[
                    pl.BlockSpec((_N_OUT, _TILE), lambda i: (0, i + tail_t0))
                ],
                core_axis_name=("c", "s"),
                dimension_semantics=(pltpu.PARALLEL,),
            )(in_hbm, out_hbm)

    return run(act_t).T


# speedup vs baseline: 20.1935x; 1.0072x over previous
import dataclasses
import functools

import jax
import jax.numpy as jnp
from jax.experimental import pallas as pl
from jax.experimental.pallas import tpu as pltpu
from jax.experimental.pallas import tpu_sc as plsc

_GROUPS = (
    (0, 1, 2, 8, 25),
    (3, 24),
    (6, 7),
    (4, 9),
    (12, 13, 14, 15),
    (16, 17, 18, 19, 20, 21, 22, 23),
)

_N_COLS = 26
_N_OUT = 6
_LANES = 16
_BLOCK_W = 1536
_TILE = 128


def _block_body(in_vmem, out_vmem):
    @plsc.parallel_loop(0, in_vmem.shape[1] // _LANES, unroll=4)
    def _(i):
        sl = pl.ds(i * _LANES, _LANES)
        for g, idx in enumerate(_GROUPS):
            m = in_vmem[idx[0], sl]
            for c in idx[1:]:
                m = jnp.maximum(m, in_vmem[c, sl])
            out_vmem[g, sl] = m


def kernel(activation):
    n_rows = activation.shape[0]
    n_tiles = -(-n_rows // _TILE)
    n_main = (n_tiles * _TILE) // _BLOCK_W
    tail_t0 = n_main * (_BLOCK_W // _TILE)
    n_tail = n_tiles - tail_t0
    act_t = activation.T
    mesh = plsc.VectorSubcoreMesh(core_axis_name="c", subcore_axis_name="s")
    cp = pltpu.CompilerParams()
    if "needs_layout_passes" in pltpu.CompilerParams.__dataclass_fields__:
        cp = dataclasses.replace(cp, needs_layout_passes=False)
    if "use_tc_tiling_on_sc" in pltpu.CompilerParams.__dataclass_fields__:
        cp = dataclasses.replace(cp, use_tc_tiling_on_sc=True)

    @functools.partial(
        pl.kernel,
        out_type=jax.ShapeDtypeStruct((_N_OUT, n_rows), jnp.float32),
        mesh=mesh,
        compiler_params=cp,
    )
    def run(in_hbm, out_hbm):
        pltpu.emit_pipeline(
            _block_body,
            grid=(n_main,),
            in_specs=[pl.BlockSpec((_N_COLS, _BLOCK_W), lambda i: (0, i))],
            out_specs=[pl.BlockSpec((_N_OUT, _BLOCK_W), lambda i: (0, i))],
            core_axis_name=("c", "s"),
            dimension_semantics=(pltpu.PARALLEL,),
        )(in_hbm, out_hbm)
        if n_tail:
            pltpu.emit_pipeline(
                _block_body,
                grid=(n_tail,),
                in_specs=[
                    pl.BlockSpec((_N_COLS, _TILE), lambda i: (0, i + tail_t0))
                ],
                out_specs=[
                    pl.BlockSpec((_N_OUT, _TILE), lambda i: (0, i + tail_t0))
                ],
                core_axis_name=("c", "s"),
                dimension_semantics=(pltpu.PARALLEL,),
            )(in_hbm, out_hbm)

    return run(act_t).T
